# baseline (speedup 1.0000x reference)
import functools

import jax
import jax.numpy as jnp
from jax import lax
from jax.experimental import pallas as pl
from jax.experimental.pallas import tpu as pltpu
from jax.experimental.pallas import tpu_sc as plsc

NUM_P = 2048
C = 19
N_IMG = 4
HW = 512 * 512
NPIX = N_IMG * HW
NW = 32
R = NPIX // NW
S = 8192
NCHUNK = R // S
TPW = NW // N_IMG
LANES = 16
SENT = 0x7FFFFFFF


def _sc_segment_stats(pred_flat, parcel_i32, target_i32):
    mesh = plsc.VectorSubcoreMesh(core_axis_name="c", subcore_axis_name="s")
    ncores = mesh.num_cores

    @functools.partial(
        pl.kernel,
        out_type=(
            jax.ShapeDtypeStruct((NW * C, NUM_P), jnp.float32),
            jax.ShapeDtypeStruct((NW, NUM_P), jnp.float32),
            jax.ShapeDtypeStruct((NW, NUM_P), jnp.int32),
        ),
        mesh=mesh,
        compiler_params=pltpu.CompilerParams(needs_layout_passes=False),
        scratch_types=[
            pltpu.VMEM((S,), jnp.int32),
            pltpu.VMEM((S,), jnp.int32),
            pltpu.VMEM((S,), jnp.float32),
            pltpu.VMEM((S,), jnp.float32),
            pltpu.VMEM((S,), jnp.float32),
            pltpu.VMEM((S,), jnp.float32),
            pltpu.VMEM((C * NUM_P,), jnp.float32),
            pltpu.VMEM((NUM_P,), jnp.float32),
            pltpu.VMEM((LANES * NUM_P,), jnp.int32),
            pltpu.VMEM((NUM_P,), jnp.int32),
            pltpu.SemaphoreType.DMA,
            pltpu.SemaphoreType.DMA,
            pltpu.SemaphoreType.DMA,
            pltpu.SemaphoreType.DMA,
        ],
    )
    def k(pred_hbm, parcel_hbm, target_hbm, sums_out, cnt_out, min_out,
          idx_v, tgt_v, val_a, val_b, val_c, val_d, acc_v, cnt_v, mint_v,
          minm_v, sem_a, sem_b, sem_c, sem_d):
        cid = lax.axis_index("c")
        sid = lax.axis_index("s")
        wid = sid * ncores + cid
        img = wid // TPW
        o0 = (wid % TPW) * R
        lanes = lax.iota(jnp.int32, LANES)
        zeros16 = jnp.zeros((LANES,), jnp.float32)
        ones16 = jnp.ones((LANES,), jnp.float32)
        sent16 = jnp.full((LANES,), SENT, jnp.int32)

        def zero_acc(i, carry):
            acc_v[pl.ds(i * LANES, LANES)] = zeros16
            return carry
        lax.fori_loop(jnp.int32(0), jnp.int32(C * NUM_P // LANES), zero_acc, 0)

        def zero_cnt(i, carry):
            cnt_v[pl.ds(i * LANES, LANES)] = zeros16
            return carry
        lax.fori_loop(jnp.int32(0), jnp.int32(NUM_P // LANES), zero_cnt, 0)

        def init_min(i, carry):
            mint_v[pl.ds(i * LANES, LANES)] = sent16
            return carry
        lax.fori_loop(jnp.int32(0), jnp.int32(LANES * NUM_P // LANES),
                      init_min, 0)

        def scatter_pair(buf0, cc0, buf1, cc1):
            acc0 = acc_v.at[pl.ds(cc0 * NUM_P, NUM_P)]
            acc1 = acc_v.at[pl.ds(cc1 * NUM_P, NUM_P)]

            @plsc.parallel_loop(jnp.int32(0), jnp.int32(S // LANES),
                                jnp.int32(1), unroll=8)
            def _(i):
                sl = pl.ds(i * LANES, LANES)
                pid = idx_v[sl]
                plsc.addupdate_scatter(acc0, [pid], buf0[sl])
                plsc.addupdate_scatter(acc1, [pid], buf1[sl])

        def scatter_one(buf, cc):
            acc_c = acc_v.at[pl.ds(cc * NUM_P, NUM_P)]

            @plsc.parallel_loop(jnp.int32(0), jnp.int32(S // LANES),
                                jnp.int32(1), unroll=8)
            def _(i):
                sl = pl.ds(i * LANES, LANES)
                plsc.addupdate_scatter(acc_c, [idx_v[sl]], buf[sl])

        bufs = (val_a, val_b, val_c, val_d)
        sems = (sem_a, sem_b, sem_c, sem_d)
        groups = [(0, 1), (2, 3), (4, 5), (6, 7), (8, 9), (10, 11),
                  (12, 13), (14, 15), (16, 17), (18,)]

        def start_group(gi, o, slot):
            cps = []
            for t, cc in enumerate(groups[gi]):
                bi = slot * 2 + t
                cps.append(pltpu.async_copy(
                    pred_hbm.at[pl.ds((img * C + cc) * HW + o, S)],
                    bufs[bi], sems[bi]))
            return cps

        def chunk_body(j, carry):
            jj = jnp.int32(NCHUNK - 1) - j
            o = o0 + jj * S
            g = img * HW + o
            pltpu.sync_copy(parcel_hbm.at[pl.ds(g, S)], idx_v)
            pltpu.sync_copy(target_hbm.at[pl.ds(g, S)], tgt_v)

            cps = {0: start_group(0, o, 0)}

            @plsc.parallel_loop(jnp.int32(0), jnp.int32(S // LANES),
                                jnp.int32(1), unroll=8)
            def _(i):
                sl = pl.ds(i * LANES, LANES)
                plsc.addupdate_scatter(cnt_v, [idx_v[sl]], ones16)

            def mc_body(i, carry2, g=g):
                ii = jnp.int32(S // LANES - 1) - i
                sl = pl.ds(ii * LANES, LANES)
                pid = idx_v[sl]
                packed = (g + ii * LANES + lanes) * 32 + tgt_v[sl]
                plsc.store_scatter(mint_v, [lanes * NUM_P + pid], packed)
                return carry2
            lax.fori_loop(jnp.int32(0), jnp.int32(S // LANES), mc_body, 0)

            for gi in range(len(groups)):
                slot = gi % 2
                for cp in cps[gi]:
                    cp.wait()
                if gi + 1 < len(groups):
                    cps[gi + 1] = start_group(gi + 1, o, 1 - slot)
                cl = groups[gi]
                if len(cl) == 2:
                    scatter_pair(bufs[slot * 2], cl[0],
                                 bufs[slot * 2 + 1], cl[1])
                else:
                    scatter_one(bufs[slot * 2], cl[0])
            return carry
        lax.fori_loop(jnp.int32(0), jnp.int32(NCHUNK), chunk_body, 0)

        def merge_body(kk, carry):
            sl0 = kk * LANES
            acc = mint_v[pl.ds(sl0, LANES)]
            for l in range(1, LANES):
                acc = jnp.minimum(acc, mint_v[pl.ds(l * NUM_P + sl0, LANES)])
            minm_v[pl.ds(sl0, LANES)] = acc
            return carry
        lax.fori_loop(jnp.int32(0), jnp.int32(NUM_P // LANES), merge_body, 0)

        for cc in range(C):
            pltpu.sync_copy(acc_v.at[pl.ds(cc * NUM_P, NUM_P)],
                            sums_out.at[wid * C + cc])
        pltpu.sync_copy(cnt_v, cnt_out.at[wid])
        pltpu.sync_copy(minm_v, min_out.at[wid])

    return k(pred_flat, parcel_i32, target_i32)


def _tc_loss(sums_parts, cnt_parts, min_parts, spc2d, sent2d):
    def body(s_ref, c_ref, m_ref, spc_ref, sent_ref, out_ref):
        s = s_ref[0:C, :]
        for w in range(1, NW):
            s = s + s_ref[w * C:(w + 1) * C, :]
        cnt = jnp.sum(c_ref[...], axis=0, keepdims=True)
        mn = jnp.min(m_ref[...], axis=0, keepdims=True)
        mn = jnp.where(mn == SENT, sent_ref[...], mn)
        lab = jnp.bitwise_and(mn, 31)
        logits = s / cnt + jnp.log(spc_ref[...])
        mx = jnp.max(logits, axis=0, keepdims=True)
        logz = jnp.log(jnp.sum(jnp.exp(logits - mx), axis=0, keepdims=True)) + mx
        oh = (lax.broadcasted_iota(jnp.int32, (C, NUM_P), 0) == lab)
        ll = jnp.sum(jnp.where(oh, logits, 0.0), axis=0, keepdims=True)
        out_ref[...] = (jnp.sum(logz - ll) / NUM_P)[None, None]

    return pl.pallas_call(
        body,
        out_shape=jax.ShapeDtypeStruct((1, 1), jnp.float32),
    )(sums_parts, cnt_parts, min_parts, spc2d, sent2d)


def kernel(pred, target, parcel, cls_num_list):
    pred_flat = pred.reshape(-1)
    parcel_i32 = parcel.reshape(512, 2048).astype(jnp.int32).reshape(-1)
    target_i32 = target.reshape(512, 2048).astype(jnp.int32).reshape(-1)
    sums_parts, cnt_parts, min_parts = _sc_segment_stats(
        pred_flat, parcel_i32, target_i32)
    spc2d = cls_num_list.astype(jnp.float32).reshape(C, 1)
    t_last = target_i32[-1]
    sent2d = ((NPIX - 1) * 32 + t_last).reshape(1, 1)
    loss32 = _tc_loss(sums_parts, cnt_parts, min_parts, spc2d, sent2d)
    return loss32[0, 0].astype(jnp.float64)


# speedup vs baseline: 1.0791x; 1.0791x over previous
import functools

import jax
import jax.numpy as jnp
from jax import lax
from jax.experimental import pallas as pl
from jax.experimental.pallas import tpu as pltpu
from jax.experimental.pallas import tpu_sc as plsc

NUM_P = 2048
C = 19
N_IMG = 4
HW = 512 * 512
NPIX = N_IMG * HW
NW = 32
R = NPIX // NW
S = 8192
NCHUNK = R // S
TPW = NW // N_IMG
LANES = 16
SENT = 0x7FFFFFFF


def _fori_unrolled(n, unroll, body):
    assert n % unroll == 0

    def big_body(i, carry):
        for u in range(unroll):
            carry = body(i * unroll + jnp.int32(u), carry)
        return carry
    return lax.fori_loop(jnp.int32(0), jnp.int32(n // unroll), big_body, 0)


def _sc_segment_stats(pred_flat, parcel_i32, target_i32):
    mesh = plsc.VectorSubcoreMesh(core_axis_name="c", subcore_axis_name="s")
    ncores = mesh.num_cores

    @functools.partial(
        pl.kernel,
        out_type=(
            jax.ShapeDtypeStruct((NW * C, NUM_P), jnp.float32),
            jax.ShapeDtypeStruct((NW, NUM_P), jnp.float32),
            jax.ShapeDtypeStruct((NW, NUM_P), jnp.int32),
        ),
        mesh=mesh,
        compiler_params=pltpu.CompilerParams(needs_layout_passes=False),
        scratch_types=[
            pltpu.VMEM((S,), jnp.int32),
            pltpu.VMEM((S,), jnp.int32),
            pltpu.VMEM((S,), jnp.float32),
            pltpu.VMEM((S,), jnp.float32),
            pltpu.VMEM((S,), jnp.float32),
            pltpu.VMEM((S,), jnp.float32),
            pltpu.VMEM((C * NUM_P,), jnp.float32),
            pltpu.VMEM((NUM_P,), jnp.float32),
            pltpu.VMEM((LANES * NUM_P,), jnp.int32),
            pltpu.VMEM((NUM_P,), jnp.int32),
            pltpu.SemaphoreType.DMA,
            pltpu.SemaphoreType.DMA,
            pltpu.SemaphoreType.DMA,
            pltpu.SemaphoreType.DMA,
        ],
    )
    def k(pred_hbm, parcel_hbm, target_hbm, sums_out, cnt_out, min_out,
          idx_v, tgt_v, val_a, val_b, val_c, val_d, acc_v, cnt_v, mint_v,
          minm_v, sem_a, sem_b, sem_c, sem_d):
        cid = lax.axis_index("c")
        sid = lax.axis_index("s")
        wid = sid * ncores + cid
        img = wid // TPW
        o0 = (wid % TPW) * R
        lanes = lax.iota(jnp.int32, LANES)
        zeros16 = jnp.zeros((LANES,), jnp.float32)
        ones16 = jnp.ones((LANES,), jnp.float32)
        sent16 = jnp.full((LANES,), SENT, jnp.int32)

        def zero_acc(i, carry):
            acc_v[pl.ds(i * LANES, LANES)] = zeros16
            return carry
        _fori_unrolled(C * NUM_P // LANES, 8, zero_acc)

        def zero_cnt(i, carry):
            cnt_v[pl.ds(i * LANES, LANES)] = zeros16
            return carry
        _fori_unrolled(NUM_P // LANES, 8, zero_cnt)

        def init_min(i, carry):
            mint_v[pl.ds(i * LANES, LANES)] = sent16
            return carry
        _fori_unrolled(LANES * NUM_P // LANES, 8, init_min)

        def scatter_pair(buf0, cc0, buf1, cc1):
            acc0 = acc_v.at[pl.ds(cc0 * NUM_P, NUM_P)]
            acc1 = acc_v.at[pl.ds(cc1 * NUM_P, NUM_P)]

            @plsc.parallel_loop(jnp.int32(0), jnp.int32(S // LANES),
                                jnp.int32(1), unroll=8)
            def _(i):
                sl = pl.ds(i * LANES, LANES)
                pid = idx_v[sl]
                plsc.addupdate_scatter(acc0, [pid], buf0[sl])
                plsc.addupdate_scatter(acc1, [pid], buf1[sl])

        def scatter_one(buf, cc):
            acc_c = acc_v.at[pl.ds(cc * NUM_P, NUM_P)]

            @plsc.parallel_loop(jnp.int32(0), jnp.int32(S // LANES),
                                jnp.int32(1), unroll=8)
            def _(i):
                sl = pl.ds(i * LANES, LANES)
                plsc.addupdate_scatter(acc_c, [idx_v[sl]], buf[sl])

        bufs = (val_a, val_b, val_c, val_d)
        sems = (sem_a, sem_b, sem_c, sem_d)
        groups = [(0, 1), (2, 3), (4, 5), (6, 7), (8, 9), (10, 11),
                  (12, 13), (14, 15), (16, 17), (18,)]

        def start_group(gi, o, slot):
            cps = []
            for t, cc in enumerate(groups[gi]):
                bi = slot * 2 + t
                cps.append(pltpu.async_copy(
                    pred_hbm.at[pl.ds((img * C + cc) * HW + o, S)],
                    bufs[bi], sems[bi]))
            return cps

        def chunk_body(j, carry):
            jj = jnp.int32(NCHUNK - 1) - j
            o = o0 + jj * S
            g = img * HW + o
            pltpu.sync_copy(parcel_hbm.at[pl.ds(g, S)], idx_v)
            pltpu.sync_copy(target_hbm.at[pl.ds(g, S)], tgt_v)

            cps = {0: start_group(0, o, 0)}

            @plsc.parallel_loop(jnp.int32(0), jnp.int32(S // LANES),
                                jnp.int32(1), unroll=8)
            def _(i):
                sl = pl.ds(i * LANES, LANES)
                plsc.addupdate_scatter(cnt_v, [idx_v[sl]], ones16)

            def mc_body(i, carry2, g=g):
                ii = jnp.int32(S // LANES - 1) - i
                sl = pl.ds(ii * LANES, LANES)
                pid = idx_v[sl]
                packed = (g + ii * LANES + lanes) * 32 + tgt_v[sl]
                plsc.store_scatter(mint_v, [lanes * NUM_P + pid], packed)
                return carry2
            _fori_unrolled(S // LANES, 8, mc_body)

            for gi in range(len(groups)):
                slot = gi % 2
                for cp in cps[gi]:
                    cp.wait()
                if gi + 1 < len(groups):
                    cps[gi + 1] = start_group(gi + 1, o, 1 - slot)
                cl = groups[gi]
                if len(cl) == 2:
                    scatter_pair(bufs[slot * 2], cl[0],
                                 bufs[slot * 2 + 1], cl[1])
                else:
                    scatter_one(bufs[slot * 2], cl[0])
            return carry
        lax.fori_loop(jnp.int32(0), jnp.int32(NCHUNK), chunk_body, 0)

        def merge_body(kk, carry):
            sl0 = kk * LANES
            acc = mint_v[pl.ds(sl0, LANES)]
            for l in range(1, LANES):
                acc = jnp.minimum(acc, mint_v[pl.ds(l * NUM_P + sl0, LANES)])
            minm_v[pl.ds(sl0, LANES)] = acc
            return carry
        lax.fori_loop(jnp.int32(0), jnp.int32(NUM_P // LANES), merge_body, 0)

        for cc in range(C):
            pltpu.sync_copy(acc_v.at[pl.ds(cc * NUM_P, NUM_P)],
                            sums_out.at[wid * C + cc])
        pltpu.sync_copy(cnt_v, cnt_out.at[wid])
        pltpu.sync_copy(minm_v, min_out.at[wid])

    return k(pred_flat, parcel_i32, target_i32)


def _tc_loss(sums_parts, cnt_parts, min_parts, spc2d, sent2d):
    def body(s_ref, c_ref, m_ref, spc_ref, sent_ref, out_ref):
        s = s_ref[0:C, :]
        for w in range(1, NW):
            s = s + s_ref[w * C:(w + 1) * C, :]
        cnt = jnp.sum(c_ref[...], axis=0, keepdims=True)
        mn = jnp.min(m_ref[...], axis=0, keepdims=True)
        mn = jnp.where(mn == SENT, sent_ref[...], mn)
        lab = jnp.bitwise_and(mn, 31)
        logits = s / cnt + jnp.log(spc_ref[...])
        mx = jnp.max(logits, axis=0, keepdims=True)
        logz = jnp.log(jnp.sum(jnp.exp(logits - mx), axis=0, keepdims=True)) + mx
        oh = (lax.broadcasted_iota(jnp.int32, (C, NUM_P), 0) == lab)
        ll = jnp.sum(jnp.where(oh, logits, 0.0), axis=0, keepdims=True)
        out_ref[...] = (jnp.sum(logz - ll) / NUM_P)[None, None]

    return pl.pallas_call(
        body,
        out_shape=jax.ShapeDtypeStruct((1, 1), jnp.float32),
    )(sums_parts, cnt_parts, min_parts, spc2d, sent2d)


def kernel(pred, target, parcel, cls_num_list):
    pred_flat = pred.reshape(-1)
    parcel_i32 = parcel.reshape(-1).astype(jnp.int32)
    target_i32 = target.reshape(-1).astype(jnp.int32)
    sums_parts, cnt_parts, min_parts = _sc_segment_stats(
        pred_flat, parcel_i32, target_i32)
    spc2d = cls_num_list.astype(jnp.float32).reshape(C, 1)
    t_last = target_i32[-1]
    sent2d = ((NPIX - 1) * 32 + t_last).reshape(1, 1)
    loss32 = _tc_loss(sums_parts, cnt_parts, min_parts, spc2d, sent2d)
    return loss32[0, 0].astype(jnp.float64)
